# baseline (speedup 1.0000x reference)
import jax
import jax.numpy as jnp
from jax.experimental import pallas as pl
from jax.experimental.pallas import tpu as pltpu

THRESH = 0.999
SQRT2 = 1.4142135623730951
BATCH = 4


def _body(tol_ref, pre_ref, gt_ref, out_ref):
    b = pl.program_id(0)

    @pl.when(b == 0)
    def _():
        out_ref[0, 0] = jnp.float32(0.0)

    tol = tol_ref[0]
    v_same = jnp.maximum(0.0 - tol, 0.0)
    v_far = jnp.maximum(SQRT2 - tol, 0.0)

    x = pre_ref[0, 0]
    y = gt_ref[0, 0]
    ca = jnp.where(x > THRESH, 1.0, 0.0)
    cb = jnp.where(y > THRESH, 1.0, 0.0)
    n_a = jnp.sum(ca)
    n_b = jnp.sum(cb)
    n_ab = jnp.sum(ca * cb)
    num_a = v_same * n_ab + v_far * (n_a - n_ab)
    num_b = v_same * n_ab + v_far * (n_b - n_ab)
    loss_b = (num_a / n_a + num_b / n_b) * (1.0 / BATCH)
    out_ref[0, 0] += loss_b


def kernel(pre, gt, tolerance):
    tol = jnp.reshape(jnp.asarray(tolerance, jnp.float32), (1,))
    out = pl.pallas_call(
        _body,
        grid=(BATCH,),
        out_shape=jax.ShapeDtypeStruct((1, 1), jnp.float32),
        in_specs=[
            pl.BlockSpec(memory_space=pltpu.SMEM),
            pl.BlockSpec((1, 1, 224, 224), lambda i: (i, 0, 0, 0)),
            pl.BlockSpec((1, 1, 224, 224), lambda i: (i, 0, 0, 0)),
        ],
        out_specs=pl.BlockSpec((1, 1), lambda i: (0, 0),
                               memory_space=pltpu.SMEM),
    )(tol, pre, gt)
    return out[0, 0]
